# Initial kernel scaffold; baseline (speedup 1.0000x reference)
import jax
import jax.numpy as jnp
from jax.experimental import pallas as pl


def kernel(positions, boxvectors, atomic_numbers, params):
    raise NotImplementedError("write your pallas kernel here")


# speedup vs baseline: 82.2547x; 82.2547x over previous
import numpy as np
import jax
import jax.numpy as jnp
from jax import lax
from jax.experimental import pallas as pl
from jax.experimental.pallas import tpu as pltpu
from jax.experimental.pallas import tpu_sc as plsc

_N = 300
_NPAD = 304
_K = 320
_KBUF = 336
_F = 128
_NG = 50
_NGP = 56
_NL = 3
_CUT = 5.0
_NW = 32
_BA = 8
_BP = _BA * _K
_NB = _NPAD // _BA
_D2PAD = 1e12
_LOG2 = float(np.log(2.0))


def _half_shifts():
    rng = np.arange(-1, 2)
    s = np.array(np.meshgrid(rng, rng, rng, indexing="ij")).reshape(3, -1).T
    nz = s[np.any(s != 0, axis=1)]
    keep = ((nz[:, 0] > 0) | ((nz[:, 0] == 0) & (nz[:, 1] > 0))
            | ((nz[:, 0] == 0) & (nz[:, 1] == 0) & (nz[:, 2] > 0)))
    return nz[keep]


_HS = _half_shifts()
_SHIFTS = np.concatenate([np.zeros((1, 3)), _HS, -_HS], axis=0).astype(np.float32)
_NS = _SHIFTS.shape[0]

_CENTERS = np.full((_NGP,), 1e9, np.float32)
_CENTERS[:_NG] = np.linspace(0.0, _CUT, _NG, dtype=np.float32)
_WIDTH = float(_CENTERS[1] - _CENTERS[0])


def _sc_body(px_h, py_h, pz_h, dx_h, dy_h, dz_h, outj_h, outd_h,
             px_v, py_v, pz_v, dx_v, dy_v, dz_v, jbuf, dbuf):
    wid = lax.axis_index("s") * 2 + lax.axis_index("c")
    pltpu.sync_copy(px_h, px_v)
    pltpu.sync_copy(py_h, py_v)
    pltpu.sync_copy(pz_h, pz_v)
    pltpu.sync_copy(dx_h, dx_v)
    pltpu.sync_copy(dy_h, dy_v)
    pltpu.sync_copy(dz_h, dz_v)
    lanes = lax.iota(jnp.int32, 16)

    def atom_body(a, carry):
        i = a * _NW + wid

        @pl.when(i < _NPAD)
        def _():
            isplat = jnp.full((16,), 0, jnp.int32) + i
            pxs = plsc.load_gather(px_v, [isplat])
            pys = plsc.load_gather(py_v, [isplat])
            pzs = plsc.load_gather(pz_v, [isplat])

            def init_body(cb, c):
                off = cb * 16
                jbuf[pl.ds(off, 16)] = jnp.zeros((16,), jnp.int32)
                dbuf[pl.ds(off, 16)] = jnp.full((16,), _D2PAD, jnp.float32)
                return c
            lax.fori_loop(0, _KBUF // 16, init_body, 0)

            def shift_body(s, wp):
                ssplat = jnp.full((16,), 0, jnp.int32) + s
                ddx = plsc.load_gather(dx_v, [ssplat])
                ddy = plsc.load_gather(dy_v, [ssplat])
                ddz = plsc.load_gather(dz_v, [ssplat])
                s_nz = ssplat != 0

                def chunk_body(c, wp2):
                    base = c * 16
                    jv = lanes + base
                    dxv = px_v[pl.ds(base, 16)] + ddx - pxs
                    dyv = py_v[pl.ds(base, 16)] + ddy - pys
                    dzv = pz_v[pl.ds(base, 16)] + ddz - pzs
                    d2 = dxv * dxv + dyv * dyv + dzv * dzv
                    m = (d2 < _CUT * _CUT) & ((jv != isplat) | s_nz)
                    plsc.store_compressed(jbuf.at[pl.ds(wp2, 16)], jv, mask=m)
                    plsc.store_compressed(dbuf.at[pl.ds(wp2, 16)], d2, mask=m)
                    return wp2 + jnp.sum(m.astype(jnp.int32))
                return lax.fori_loop(0, _NPAD // 16, chunk_body, wp)

            lax.fori_loop(0, _NS, shift_body, 0)
            pltpu.sync_copy(jbuf.at[pl.ds(0, _K)], outj_h.at[pl.ds(i * _K, _K)])
            pltpu.sync_copy(dbuf.at[pl.ds(0, _K)], outd_h.at[pl.ds(i * _K, _K)])
        return carry

    lax.fori_loop(0, (_NPAD + _NW - 1) // _NW, atom_body, 0)


def _sc_mesh():
    return plsc.VectorSubcoreMesh(core_axis_name="c", subcore_axis_name="s")


_SC_OUT = (jax.ShapeDtypeStruct((_NPAD * _K,), jnp.int32),
           jax.ShapeDtypeStruct((_NPAD * _K,), jnp.float32))
_SC_SCRATCH = [
    pltpu.VMEM((_NPAD,), jnp.float32),
    pltpu.VMEM((_NPAD,), jnp.float32),
    pltpu.VMEM((_NPAD,), jnp.float32),
    pltpu.VMEM((32,), jnp.float32),
    pltpu.VMEM((32,), jnp.float32),
    pltpu.VMEM((32,), jnp.float32),
    pltpu.VMEM((_KBUF,), jnp.int32),
    pltpu.VMEM((_KBUF,), jnp.float32),
]


def _ssp(x):
    return jnp.maximum(x, 0.0) + jnp.log1p(jnp.exp(-jnp.abs(x))) - _LOG2


def _tc_body(d2_ref, nbh_ref, cen_ref, anoh_ref, emb_ref,
             win_ref, wf1_ref, bf1_ref, wf2_ref, bf2_ref,
             wfo_ref, bfo_ref, wd_ref, bd_ref,
             ow1_ref, ob1_ref, ow2t_ref, ob2_ref,
             out_ref, x_s, y_s):
    l = pl.program_id(0)
    b = pl.program_id(1)

    @pl.when((l == 0) & (b == 0))
    def _init():
        x_s[...] = jnp.dot(anoh_ref[...], emb_ref[...],
                           preferred_element_type=jnp.float32)

    @pl.when(b == 0)
    def _snapshot():
        y_s[...] = jnp.dot(x_s[...], win_ref[0],
                           preferred_element_type=jnp.float32)

    d2 = d2_ref[...]
    r = jnp.sqrt(d2 + 1e-12)
    diff = (r - cen_ref[...]) / _WIDTH
    g = jnp.exp(-0.5 * diff * diff)
    h = _ssp(jnp.dot(g, wf1_ref[0], preferred_element_type=jnp.float32)
             + bf1_ref[0])
    w = jnp.dot(h, wf2_ref[0], preferred_element_type=jnp.float32) + bf2_ref[0]
    incut = (r < _CUT).astype(jnp.float32)
    t = jnp.pi * jnp.clip(r * (1.0 / _CUT) - 0.5, -0.5, 0.5)
    t2 = t * t
    sin_t = t * (1.0 + t2 * (-1.0 / 6.0 + t2 * (1.0 / 120.0 + t2 * (
        -1.0 / 5040.0 + t2 * (1.0 / 362880.0 + t2 * (-1.0 / 39916800.0))))))
    fcut = 0.5 * (1.0 - sin_t) * incut
    w = w * fcut

    nbh = nbh_ref[...]
    cols = lax.broadcasted_iota(jnp.int32, (_BP, _NPAD), 1)
    oh = (nbh == cols).astype(jnp.float32)
    yj = jnp.dot(oh, y_s[...], preferred_element_type=jnp.float32)
    z = yj * w

    q = lax.broadcasted_iota(jnp.int32, (_BA, _BP), 1)
    arow = lax.broadcasted_iota(jnp.int32, (_BA, _BP), 0)
    seg = ((q >= arow * _K) & (q < (arow + 1) * _K)).astype(jnp.float32)
    agg = jnp.dot(seg, z, preferred_element_type=jnp.float32)

    v = _ssp(jnp.dot(agg, wfo_ref[0], preferred_element_type=jnp.float32)
             + bfo_ref[0])
    v = jnp.dot(v, wd_ref[0], preferred_element_type=jnp.float32) + bd_ref[0]
    rows = pl.ds(b * _BA, _BA)
    x_s[rows, :] = x_s[rows, :] + v

    @pl.when((l == _NL - 1) & (b == _NB - 1))
    def _energy():
        x = x_s[...]
        yi = _ssp(jnp.dot(x, ow1_ref[...], preferred_element_type=jnp.float32)
                  + ob1_ref[...])
        e = jnp.sum(yi * ow2t_ref[...], axis=1, keepdims=True) + ob2_ref[...]
        amask = (lax.broadcasted_iota(jnp.int32, (_NPAD, 1), 0) < _N)
        out_ref[...] = jnp.sum(jnp.where(amask, e, 0.0), keepdims=True)


def _w_idx(l, b):
    return (l, 0, 0)


def _z2_idx(l, b):
    return (0, 0)


_TC_KWARGS = dict(
    grid=(_NL, _NB),
    in_specs=[
        pl.BlockSpec((_BP, 1), lambda l, b: (b, 0)),
        pl.BlockSpec((_BP, 1), lambda l, b: (b, 0)),
        pl.BlockSpec((1, _NGP), _z2_idx),
        pl.BlockSpec((_NPAD, 100), _z2_idx),
        pl.BlockSpec((100, _F), _z2_idx),
        pl.BlockSpec((1, _F, _F), _w_idx),
        pl.BlockSpec((1, _NGP, _F), _w_idx),
        pl.BlockSpec((1, 1, _F), _w_idx),
        pl.BlockSpec((1, _F, _F), _w_idx),
        pl.BlockSpec((1, 1, _F), _w_idx),
        pl.BlockSpec((1, _F, _F), _w_idx),
        pl.BlockSpec((1, 1, _F), _w_idx),
        pl.BlockSpec((1, _F, _F), _w_idx),
        pl.BlockSpec((1, 1, _F), _w_idx),
        pl.BlockSpec((_F, 64), _z2_idx),
        pl.BlockSpec((1, 64), _z2_idx),
        pl.BlockSpec((1, 64), _z2_idx),
        pl.BlockSpec((1, 1), _z2_idx),
    ],
    out_specs=pl.BlockSpec((1, 1), _z2_idx),
    out_shape=jax.ShapeDtypeStruct((1, 1), jnp.float32),
    scratch_shapes=[pltpu.VMEM((_NPAD, _F), jnp.float32),
                    pltpu.VMEM((_NPAD, _F), jnp.float32)],
    compiler_params=pltpu.CompilerParams(
        dimension_semantics=("arbitrary", "arbitrary")),
)


def _sc_inputs(positions, boxvectors):
    pos10 = positions.astype(jnp.float32) * 10.0
    cell10 = boxvectors.astype(jnp.float32) * 10.0
    padx = jnp.asarray([1e6, 2e6, 3e6, 4e6], jnp.float32)
    pad0 = jnp.zeros((_NPAD - _N,), jnp.float32)
    px = jnp.concatenate([pos10[:, 0], padx])
    py = jnp.concatenate([pos10[:, 1], pad0])
    pz = jnp.concatenate([pos10[:, 2], pad0])
    disp = jnp.concatenate(
        [jnp.asarray(_SHIFTS) @ cell10, jnp.zeros((32 - _NS, 3), jnp.float32)])
    return (px, py, pz,
            disp[:, 0] + 0.0,
            disp[:, 1] + 0.0,
            disp[:, 2] + 0.0)


def _tc_inputs(d2, nbh, atomic_numbers, params):
    d2col = d2.reshape(_NPAD * _K, 1)
    nbhcol = nbh.reshape(_NPAD * _K, 1)
    an = jnp.concatenate([jnp.asarray(atomic_numbers, jnp.int32),
                          jnp.full((_NPAD - _N,), -1, jnp.int32)])
    anoh = (an[:, None] == jnp.arange(100, dtype=jnp.int32)[None, :]
            ).astype(jnp.float32)
    inter = params["interactions"]
    win = jnp.stack([p["in2f_W"] for p in inter])
    wf1 = jnp.stack([jnp.pad(p["f_W1"], ((0, _NGP - _NG), (0, 0)))
                     for p in inter])
    bf1 = jnp.stack([p["f_b1"] for p in inter]).reshape(_NL, 1, _F)
    wf2 = jnp.stack([p["f_W2"] for p in inter])
    bf2 = jnp.stack([p["f_b2"] for p in inter]).reshape(_NL, 1, _F)
    wfo = jnp.stack([p["f2out_W"] for p in inter])
    bfo = jnp.stack([p["f2out_b"] for p in inter]).reshape(_NL, 1, _F)
    wd = jnp.stack([p["dense_W"] for p in inter])
    bd = jnp.stack([p["dense_b"] for p in inter]).reshape(_NL, 1, _F)
    ow1 = params["out_W1"]
    ob1 = params["out_b1"].reshape(1, 64)
    ow2t = params["out_W2"].reshape(1, 64)
    ob2 = params["out_b2"].reshape(1, 1)
    centers = jnp.concatenate([jnp.linspace(0.0, _CUT, _NG),
                               jnp.full((_NGP - _NG,), 1e9, jnp.float32)])
    cen = centers.astype(jnp.float32).reshape(1, _NGP)
    return (d2col, nbhcol, cen, anoh, params["embedding"],
            win, wf1, bf1, wf2, bf2, wfo, bfo, wd, bd,
            ow1, ob1, ow2t, ob2)


def kernel(positions, boxvectors, atomic_numbers, params):
    sc = pl.kernel(_sc_body, out_type=_SC_OUT, mesh=_sc_mesh(),
                   scratch_types=_SC_SCRATCH,
                   compiler_params=pltpu.CompilerParams(
                       needs_layout_passes=False))
    nbh, d2 = sc(*_sc_inputs(positions, boxvectors))
    out = pl.pallas_call(_tc_body, **_TC_KWARGS)(
        *_tc_inputs(d2, nbh, atomic_numbers, params))
    return out[0, 0]
